# baseline (speedup 1.0000x reference)
import functools
import math

import jax
import jax.numpy as jnp
from jax import lax
from jax.experimental import pallas as pl
from jax.experimental.pallas import tpu as pltpu
from jax.experimental.pallas import tpu_sc as plsc

ALPHA = 0.4
BETA = math.log(0.5 + 1.0)

H = 128
HW = H // 2


def _make_sc_pass(n_src, n_dst, E, packed):
    K, NG, ZROWS = 40, 10, 40
    info = plsc.get_sparse_core_info()
    NC, NS = info.num_cores, info.num_subcores
    NW = NC * NS
    per_w = E // NW
    assert per_w * NW == E and per_w % K == 0
    n_chunks = per_w // K
    G = n_chunks // NG
    assert G * NG == n_chunks
    P = G // 2
    tail = G % 2
    nz_chunks = n_dst // ZROWS
    assert nz_chunks * ZROWS == n_dst and ZROWS <= K
    z_iters = (nz_chunks + NS - 1) // NS
    SROWS = 40
    nt_chunks = n_src // SROWS
    assert nt_chunks * SROWS == n_src
    t_iters = (nt_chunks + NS - 1) // NS

    mesh = plsc.VectorSubcoreMesh(core_axis_name="c", subcore_axis_name="s")

    scratch = [
        pltpu.VMEM((2, G, K), jnp.int32),
        pltpu.VMEM((2, G, K), jnp.int32),
        pltpu.VMEM((2, G, K), jnp.float32),
        pltpu.VMEM((2, K, HW) if packed else (2, K, H),
                   jnp.int32 if packed else jnp.float32),
        pltpu.VMEM((2, K, H), jnp.float32) if packed else None,
        pltpu.VMEM_SHARED((n_src, HW), jnp.int32) if packed else None,
        pltpu.VMEM_SHARED((n_dst, H), jnp.float32),
        [pltpu.SemaphoreType.DMA] * 2,
        [pltpu.SemaphoreType.DMA] * 2,
        pltpu.SemaphoreType.DMA,
        pltpu.SemaphoreType.DMA,
    ]
    scratch = [x for x in scratch if x is not None]

    def _build(body):
        return functools.partial(
            pl.kernel,
            out_type=jax.ShapeDtypeStruct((NC, n_dst, H), jnp.float32),
            mesh=mesh,
            compiler_params=pltpu.CompilerParams(needs_layout_passes=False),
            scratch_types=scratch,
        )(body)

    def sc_pass(*refs):
        if packed:
            (table, sidx, didx, w, out, sidx_v, didx_v, w_v,
             rows, frows, shtab, acc, semg2, sems2, semstg, semtab) = refs
        else:
            (table, sidx, didx, w, out, sidx_v, didx_v, w_v,
             rows, acc, semg2, sems2, semstg, semtab) = refs
            frows = rows
        c = lax.axis_index("c")
        s = lax.axis_index("s")
        wid = s * NC + c

        if packed:
            def _stage_tab(k, _):
                ti = s + k * NS
                @pl.when(ti < nt_chunks)
                def _():
                    pltpu.async_copy(table.at[pl.ds(ti * SROWS, SROWS)],
                                     shtab.at[pl.ds(ti * SROWS, SROWS)],
                                     semtab)
                return 0
            lax.fori_loop(0, t_iters, _stage_tab, 0)

        def _zero_row(i, _):
            for t in range(H // 16):
                frows[0, i, pl.ds(t * 16, 16)] = jnp.zeros((16,), jnp.float32)
            return 0
        lax.fori_loop(0, ZROWS, _zero_row, 0)

        def _zero_acc(k, _):
            zi = s + k * NS
            @pl.when(zi < nz_chunks)
            def _():
                pltpu.sync_copy(frows.at[0, pl.ds(0, ZROWS)],
                                acc.at[pl.ds(zi * ZROWS, ZROWS)])
            return 0
        lax.fori_loop(0, z_iters, _zero_acc, 0)

        if packed:
            def _drain_tab(k, _):
                ti = s + k * NS
                @pl.when(ti < nt_chunks)
                def _():
                    pltpu.make_async_copy(
                        table.at[pl.ds(ti * SROWS, SROWS)],
                        shtab.at[pl.ds(ti * SROWS, SROWS)], semtab).wait()
                return 0
            lax.fori_loop(0, t_iters, _drain_tab, 0)
        plsc.subcore_barrier()

        def _stage_group(g, slot):
            pltpu.async_copy(sidx.at[wid, g], sidx_v.at[slot], semstg)
            pltpu.async_copy(didx.at[wid, g], didx_v.at[slot], semstg)
            pltpu.async_copy(w.at[wid, g], w_v.at[slot], semstg)

        def _wait_stage(g, slot):
            pltpu.make_async_copy(sidx.at[wid, g], sidx_v.at[slot],
                                  semstg).wait()
            pltpu.make_async_copy(didx.at[wid, g], didx_v.at[slot],
                                  semstg).wait()
            pltpu.make_async_copy(w.at[wid, g], w_v.at[slot], semstg).wait()

        gsrc = shtab if packed else table

        def _fire_g(slot, j, b):
            pltpu.async_copy(gsrc.at[sidx_v.at[slot, j]], rows.at[b],
                             semg2[b])

        def _wait_g(slot, j, b):
            pltpu.make_async_copy(gsrc.at[sidx_v.at[slot, j]], rows.at[b],
                                  semg2[b]).wait()

        def _fire_s(slot, j, b):
            pltpu.async_copy(frows.at[b], acc.at[didx_v.at[slot, j]],
                             sems2[b], add=True)

        def _wait_s(slot, j, b):
            pltpu.make_async_copy(frows.at[b], acc.at[didx_v.at[slot, j]],
                                  sems2[b]).wait()

        def _scale(slot, j, b):
            w_row = w_v.at[slot, j]

            if packed:
                @plsc.parallel_loop(0, K, 1, unroll=4)
                def _row(i):
                    wspl = plsc.load_gather(
                        w_row, [jnp.full((16,), i, jnp.int32)])
                    for t in range(H // 32):
                        x = rows[b, i, pl.ds(t * 16, 16)]
                        lo = plsc.bitcast(jnp.left_shift(x, 16),
                                          jnp.float32)
                        hi = plsc.bitcast(
                            jnp.bitwise_and(x, jnp.int32(-65536)),
                            jnp.float32)
                        frows[b, i, pl.ds(t * 32, 16)] = lo * wspl
                        frows[b, i, pl.ds(t * 32 + 16, 16)] = hi * wspl
            else:
                @plsc.parallel_loop(0, K, 1, unroll=4)
                def _row(i):
                    wspl = plsc.load_gather(
                        w_row, [jnp.full((16,), i, jnp.int32)])
                    for t in range(H // 16):
                        sl = pl.ds(t * 16, 16)
                        rows[b, i, sl] = rows[b, i, sl] * wspl

        _stage_group(0, 0)
        _wait_stage(0, 0)

        def _group(g, _):
            slot = g % 2

            @pl.when(g + 1 < NG)
            def _():
                _stage_group(g + 1, 1 - slot)

            _fire_g(slot, 0, 0)

            def _pair(q, _):
                j0 = 2 * q
                j1 = j0 + 1
                _wait_g(slot, j0, 0)

                @pl.when(q > 0)
                def _():
                    _wait_s(slot, j1 - 2, 1)
                _fire_g(slot, j1, 1)

                if packed:
                    @pl.when(q > 0)
                    def _():
                        _wait_s(slot, j0 - 2, 0)
                _scale(slot, j0, 0)
                _fire_s(slot, j0, 0)
                _wait_g(slot, j1, 1)
                if not packed:
                    _wait_s(slot, j0, 0)

                @pl.when(j1 + 1 < G)
                def _():
                    _fire_g(slot, j1 + 1, 0)
                _scale(slot, j1, 1)
                _fire_s(slot, j1, 1)
                return 0
            lax.fori_loop(0, P, _pair, 0)

            if tail:
                jt = G - 1
                _wait_g(slot, jt, 0)
                if packed:
                    _wait_s(slot, jt - 2, 0)
                _scale(slot, jt, 0)
                _fire_s(slot, jt, 0)
                _wait_s(slot, jt - 1, 1)
                _wait_s(slot, jt, 0)
            else:
                if packed:
                    _wait_s(slot, G - 2, 0)
                _wait_s(slot, G - 1, 1)

            @pl.when(g + 1 < NG)
            def _():
                _wait_stage(g + 1, 1 - slot)
            return 0
        lax.fori_loop(0, NG, _group, 0)

        plsc.subcore_barrier()

        def _writeback(k, _):
            zi = s + k * NS
            @pl.when(zi < nz_chunks)
            def _():
                pltpu.sync_copy(acc.at[pl.ds(zi * ZROWS, ZROWS)],
                                out.at[c, pl.ds(zi * ZROWS, ZROWS)])
            return 0
        lax.fori_loop(0, z_iters, _writeback, 0)

    return _build(sc_pass)


def _row_specs(n_rows, blk, n_extra_full):
    return pl.BlockSpec((blk, H), lambda i: (i, 0))


def _tc_call(body, grid, in_specs, out_specs, out_shape, args):
    return pl.pallas_call(
        body, grid=grid, in_specs=in_specs, out_specs=out_specs,
        out_shape=out_shape)(*args)


def _full2d(a, b):
    return pl.BlockSpec((a, b), lambda i: (0, 0))


def _stage1(v, W_vtx, b_vtx, W0, b0, nw, blk):
    n = v.shape[0]

    def body(v_ref, Wv_ref, bv_ref, W0_ref, b0_ref, nw_ref, vA_ref, ves_ref):
        v1 = jnp.dot(v_ref[...], Wv_ref[...],
                     preferred_element_type=jnp.float32) + bv_ref[...]
        nwb = nw_ref[...]
        vA_ref[...] = v1 * nwb
        ve = jnp.maximum(jnp.dot(v1, W0_ref[...],
                                 preferred_element_type=jnp.float32)
                         + b0_ref[...], 0.0)
        ves_ref[...] = ve * nwb

    return _tc_call(
        body, (n // blk,),
        [_row_specs(n, blk, 0), _full2d(H, H), _full2d(1, H),
         _full2d(H, H), _full2d(1, H),
         pl.BlockSpec((blk, 1), lambda i: (i, 0))],
        [_row_specs(n, blk, 0)] * 2,
        [jax.ShapeDtypeStruct((n, H), jnp.float32)] * 2,
        (v, W_vtx, b_vtx, W0, b0, nw))


def _stage2(e, eacc, ers, W, b, ew, blk):
    n = e.shape[0]

    def body(e_ref, acc_ref, ers_ref, W_ref, b_ref, ew_ref, e1_ref, evs_ref):
        a = acc_ref[...]
        e1 = (e_ref[...] + a[0] + a[1]) / ers_ref[...]
        e1_ref[...] = e1
        ev = jnp.maximum(jnp.dot(e1, W_ref[...],
                                 preferred_element_type=jnp.float32)
                         + b_ref[...], 0.0)
        evs_ref[...] = ev * ew_ref[...]

    return _tc_call(
        body, (n // blk,),
        [_row_specs(n, blk, 0),
         pl.BlockSpec((2, blk, H), lambda i: (0, i, 0)),
         pl.BlockSpec((blk, 1), lambda i: (i, 0)),
         _full2d(H, H), _full2d(1, H),
         pl.BlockSpec((blk, 1), lambda i: (i, 0))],
        [_row_specs(n, blk, 0)] * 2,
        [jax.ShapeDtypeStruct((n, H), jnp.float32)] * 2,
        (e, eacc, ers, W, b, ew))


def _stage3(vA, vacc, nrs, W1, b1, nw, blk):
    n = vA.shape[0]

    def body(vA_ref, acc_ref, nrs_ref, W_ref, b_ref, nw_ref,
             v2_ref, vB_ref, ve2s_ref):
        a = acc_ref[...]
        v2 = (vA_ref[...] + a[0] + a[1]) / nrs_ref[...]
        v2_ref[...] = v2
        nwb = nw_ref[...]
        vB_ref[...] = v2 * nwb
        ve2 = jnp.maximum(
            (1.0 - BETA) * (jnp.dot(v2, W_ref[...],
                                    preferred_element_type=jnp.float32)
                            + b_ref[...]) + BETA * v2, 0.0)
        ve2s_ref[...] = ve2 * nwb

    return _tc_call(
        body, (n // blk,),
        [_row_specs(n, blk, 0),
         pl.BlockSpec((2, blk, H), lambda i: (0, i, 0)),
         pl.BlockSpec((blk, 1), lambda i: (i, 0)),
         _full2d(H, H), _full2d(1, H),
         pl.BlockSpec((blk, 1), lambda i: (i, 0))],
        [_row_specs(n, blk, 0)] * 3,
        [jax.ShapeDtypeStruct((n, H), jnp.float32)] * 3,
        (vA, vacc, nrs, W1, b1, nw))


def _stage4(e1, eacc2, ers, W, b, ew, blk):
    n = e1.shape[0]

    def body(e1_ref, acc_ref, ers_ref, W_ref, b_ref, ew_ref,
             e2_ref, ev2s_ref):
        a = acc_ref[...]
        e1 = e1_ref[...]
        e2a = (e1 + a[0] + a[1]) / ers_ref[...]
        e2 = (1.0 - ALPHA) * e2a + ALPHA * e1
        e2_ref[...] = e2
        ev2 = jnp.maximum(
            (1.0 - BETA) * (jnp.dot(e2, W_ref[...],
                                    preferred_element_type=jnp.float32)
                            + b_ref[...]) + BETA * e2, 0.0)
        ev2s_ref[...] = ev2 * ew_ref[...]

    return _tc_call(
        body, (n // blk,),
        [_row_specs(n, blk, 0),
         pl.BlockSpec((2, blk, H), lambda i: (0, i, 0)),
         pl.BlockSpec((blk, 1), lambda i: (i, 0)),
         _full2d(H, H), _full2d(1, H),
         pl.BlockSpec((blk, 1), lambda i: (i, 0))],
        [_row_specs(n, blk, 0)] * 2,
        [jax.ShapeDtypeStruct((n, H), jnp.float32)] * 2,
        (e1, eacc2, ers, W, b, ew))


def _stage5(vB, vacc2, nrs, v2, W_cls, b_cls, blk):
    n = vB.shape[0]
    ncls = W_cls.shape[1]

    def body(vB_ref, acc_ref, nrs_ref, v2_ref, W_ref, b_ref,
             vout_ref, pred_ref):
        a = acc_ref[...]
        v3 = (vB_ref[...] + a[0] + a[1]) / nrs_ref[...]
        vout = (1.0 - ALPHA) * v3 + ALPHA * v2_ref[...]
        vout_ref[...] = vout
        pred_ref[...] = jnp.dot(vout, W_ref[...],
                                preferred_element_type=jnp.float32) + b_ref[...]

    return _tc_call(
        body, (n // blk,),
        [_row_specs(n, blk, 0),
         pl.BlockSpec((2, blk, H), lambda i: (0, i, 0)),
         pl.BlockSpec((blk, 1), lambda i: (i, 0)),
         _row_specs(n, blk, 0),
         _full2d(H, ncls), _full2d(1, ncls)],
        [_row_specs(n, blk, 0), pl.BlockSpec((blk, ncls), lambda i: (i, 0))],
        [jax.ShapeDtypeStruct((n, H), jnp.float32),
         jax.ShapeDtypeStruct((n, ncls), jnp.float32)],
        (vB, vacc2, nrs, v2, W_cls, b_cls))


def kernel(v, e, W_vtx, b_vtx, W_v2e0, b_v2e0, W_e2v0, b_e2v0,
           W_v2e1, b_v2e1, W_e2v1, b_e2v1, W_cls, b_cls,
           vidx, eidx, n_weight, e_weight, n_reg_weight, e_reg_weight,
           n_reg_sum, e_reg_sum):
    NV = v.shape[0]
    NE = e.shape[0]
    E = vidx.shape[0]

    info = plsc.get_sparse_core_info()
    NW = info.num_cores * info.num_subcores
    KC, NGC = 40, 10
    vidx2 = vidx.reshape(NW, NGC, -1, KC)
    eidx2 = eidx.reshape(NW, NGC, -1, KC)
    nrw2 = n_reg_weight.reshape(NW, NGC, -1, KC)
    erw2 = e_reg_weight.reshape(NW, NGC, -1, KC)
    b_vtx2 = b_vtx.reshape(1, H)
    b_v2e0_2 = b_v2e0.reshape(1, H)
    b_e2v0_2 = b_e2v0.reshape(1, H)
    b_v2e1_2 = b_v2e1.reshape(1, H)
    b_e2v1_2 = b_e2v1.reshape(1, H)
    b_cls2 = b_cls.reshape(1, -1)

    blk_v = 1000
    blk_e = 1000

    sc_v2e = _make_sc_pass(NV, NE, E, packed=False)
    sc_e2v = _make_sc_pass(NE, NV, E, packed=False)

    def _pack(t):
        n = t.shape[0]
        return lax.bitcast_convert_type(t.reshape(n, HW, 2), jnp.int32)

    def _unperm(p):
        n = p.shape[1]
        return p.reshape(2, n, H // 32, 2, 16).swapaxes(3, 4).reshape(
            2, n, H)

    vA, ves = _stage1(v, W_vtx, b_vtx2, W_v2e0, b_v2e0_2, n_weight, blk_v)
    eacc = sc_v2e(ves, vidx2, eidx2, nrw2)
    e1, evs = _stage2(e, eacc, e_reg_sum, W_e2v0, b_e2v0_2, e_weight, blk_e)
    vacc = sc_e2v(evs, eidx2, vidx2, erw2)
    v2, vB, ve2s = _stage3(vA, vacc, n_reg_sum, W_v2e1, b_v2e1_2,
                           n_weight, blk_v)
    eacc2 = sc_v2e(ve2s, vidx2, eidx2, nrw2)
    e2, ev2s = _stage4(e1, eacc2, e_reg_sum, W_e2v1, b_e2v1_2,
                       e_weight, blk_e)
    vacc2 = sc_e2v(ev2s, eidx2, vidx2, erw2)
    v_out, pred = _stage5(vB, vacc2, n_reg_sum, v2, W_cls, b_cls2, blk_v)

    return (v_out, e2, pred)


# speedup vs baseline: 1.4411x; 1.4411x over previous
import functools
import math

import jax
import jax.numpy as jnp
from jax import lax
from jax.experimental import pallas as pl
from jax.experimental.pallas import tpu as pltpu
from jax.experimental.pallas import tpu_sc as plsc

ALPHA = 0.4
BETA = math.log(0.5 + 1.0)

H = 128
HW = H // 2


K = 40
NG = 10
ZROWS = 40


def _make_sc_pass(n_src, n_dst, E, packed=False):
    info = plsc.get_sparse_core_info()
    NC, NS = info.num_cores, info.num_subcores
    NW = NC * NS
    per_w = E // NW
    assert per_w * NW == E and per_w % K == 0
    n_chunks = per_w // K
    G = n_chunks // NG
    assert G * NG == n_chunks and G % 3 == 1
    Q = G // 3
    nz_chunks = n_dst // ZROWS
    assert nz_chunks * ZROWS == n_dst
    z_iters = (nz_chunks + NS - 1) // NS

    mesh = plsc.VectorSubcoreMesh(core_axis_name="c", subcore_axis_name="s")

    @functools.partial(
        pl.kernel,
        out_type=jax.ShapeDtypeStruct((NC, n_dst, H), jnp.float32),
        mesh=mesh,
        compiler_params=pltpu.CompilerParams(needs_layout_passes=False),
        scratch_types=[
            pltpu.VMEM((2, G, K), jnp.int32),
            pltpu.VMEM((2, G, K), jnp.int32),
            pltpu.VMEM((2, G, K), jnp.float32),
            pltpu.VMEM((3, K, H), jnp.float32),
            pltpu.VMEM_SHARED((n_dst, H), jnp.float32),
            [pltpu.SemaphoreType.DMA] * 3,
            [pltpu.SemaphoreType.DMA] * 3,
            pltpu.SemaphoreType.DMA,
        ],
    )
    def sc_pass(table, sidx, didx, w, out, sidx_v, didx_v, w_v,
                rows, acc, semg3, sems3, semstg):
        c = lax.axis_index("c")
        s = lax.axis_index("s")
        wid = s * NC + c

        def _zero_row(i, _):
            for t in range(H // 16):
                rows[0, i, pl.ds(t * 16, 16)] = jnp.zeros((16,), jnp.float32)
            return 0
        lax.fori_loop(0, ZROWS, _zero_row, 0)

        def _zero_acc(k, _):
            zi = s + k * NS
            @pl.when(zi < nz_chunks)
            def _():
                pltpu.sync_copy(rows.at[0, pl.ds(0, ZROWS)],
                                acc.at[pl.ds(zi * ZROWS, ZROWS)])
            return 0
        lax.fori_loop(0, z_iters, _zero_acc, 0)
        plsc.subcore_barrier()

        def _stage_group(g, slot):
            pltpu.async_copy(sidx.at[wid, g], sidx_v.at[slot], semstg)
            pltpu.async_copy(didx.at[wid, g], didx_v.at[slot], semstg)
            pltpu.async_copy(w.at[wid, g], w_v.at[slot], semstg)

        def _wait_stage(g, slot):
            pltpu.make_async_copy(sidx.at[wid, g], sidx_v.at[slot],
                                  semstg).wait()
            pltpu.make_async_copy(didx.at[wid, g], didx_v.at[slot],
                                  semstg).wait()
            pltpu.make_async_copy(w.at[wid, g], w_v.at[slot], semstg).wait()

        def _fire_g(slot, j, b):
            pltpu.async_copy(table.at[sidx_v.at[slot, j]], rows.at[b],
                             semg3[b])

        def _wait_g(slot, j, b):
            pltpu.make_async_copy(table.at[sidx_v.at[slot, j]], rows.at[b],
                                  semg3[b]).wait()

        def _fire_s(slot, j, b):
            pltpu.async_copy(rows.at[b], acc.at[didx_v.at[slot, j]],
                             sems3[b], add=True)

        def _wait_s(slot, j, b):
            pltpu.make_async_copy(rows.at[b], acc.at[didx_v.at[slot, j]],
                                  sems3[b]).wait()

        def _scale(slot, j, b):
            w_row = w_v.at[slot, j]

            @plsc.parallel_loop(0, K, 1, unroll=4)
            def _row(i):
                wspl = plsc.load_gather(
                    w_row, [jnp.full((16,), i, jnp.int32)])
                for t in range(H // 16):
                    sl = pl.ds(t * 16, 16)
                    rows[b, i, sl] = rows[b, i, sl] * wspl

        _stage_group(0, 0)
        _wait_stage(0, 0)

        def _group(g, _):
            slot = g % 2

            @pl.when(g + 1 < NG)
            def _():
                _stage_group(g + 1, 1 - slot)

            _fire_g(slot, 0, 0)
            _fire_g(slot, 1, 1)

            def _triple(q, _):
                j0 = 3 * q
                j1 = j0 + 1
                j2 = j0 + 2

                @pl.when(q > 0)
                def _():
                    _wait_s(slot, j0 - 1, 2)
                _fire_g(slot, j2, 2)
                _wait_g(slot, j0, 0)
                _scale(slot, j0, 0)
                _fire_s(slot, j0, 0)
                _wait_g(slot, j1, 1)
                _scale(slot, j1, 1)
                _fire_s(slot, j1, 1)
                _wait_s(slot, j0, 0)
                _fire_g(slot, j0 + 3, 0)
                _wait_g(slot, j2, 2)
                _scale(slot, j2, 2)
                _fire_s(slot, j2, 2)
                _wait_s(slot, j1, 1)

                @pl.when(j1 + 3 < G)
                def _():
                    _fire_g(slot, j1 + 3, 1)
                return 0
            lax.fori_loop(0, Q, _triple, 0)

            jt = 3 * Q
            _wait_s(slot, jt - 1, 2)
            _wait_g(slot, jt, 0)
            _scale(slot, jt, 0)
            _fire_s(slot, jt, 0)
            _wait_s(slot, jt, 0)

            @pl.when(g + 1 < NG)
            def _():
                _wait_stage(g + 1, 1 - slot)
            return 0
        lax.fori_loop(0, NG, _group, 0)

        plsc.subcore_barrier()

        def _writeback(k, _):
            zi = s + k * NS
            @pl.when(zi < nz_chunks)
            def _():
                pltpu.sync_copy(acc.at[pl.ds(zi * ZROWS, ZROWS)],
                                out.at[c, pl.ds(zi * ZROWS, ZROWS)])
            return 0
        lax.fori_loop(0, z_iters, _writeback, 0)

    return sc_pass


def _row_specs(n_rows, blk, n_extra_full):
    return pl.BlockSpec((blk, H), lambda i: (i, 0))


def _tc_call(body, grid, in_specs, out_specs, out_shape, args):
    return pl.pallas_call(
        body, grid=grid, in_specs=in_specs, out_specs=out_specs,
        out_shape=out_shape)(*args)


def _full2d(a, b):
    return pl.BlockSpec((a, b), lambda i: (0, 0))


def _stage1(v, W_vtx, b_vtx, W0, b0, nw, blk):
    n = v.shape[0]

    def body(v_ref, Wv_ref, bv_ref, W0_ref, b0_ref, nw_ref, vA_ref, ves_ref):
        v1 = jnp.dot(v_ref[...], Wv_ref[...],
                     preferred_element_type=jnp.float32) + bv_ref[...]
        nwb = nw_ref[...]
        vA_ref[...] = v1 * nwb
        ve = jnp.maximum(jnp.dot(v1, W0_ref[...],
                                 preferred_element_type=jnp.float32)
                         + b0_ref[...], 0.0)
        ves_ref[...] = ve * nwb

    return _tc_call(
        body, (n // blk,),
        [_row_specs(n, blk, 0), _full2d(H, H), _full2d(1, H),
         _full2d(H, H), _full2d(1, H),
         pl.BlockSpec((blk, 1), lambda i: (i, 0))],
        [_row_specs(n, blk, 0)] * 2,
        [jax.ShapeDtypeStruct((n, H), jnp.float32)] * 2,
        (v, W_vtx, b_vtx, W0, b0, nw))


def _stage2(e, eacc, ers, W, b, ew, blk):
    n = e.shape[0]

    def body(e_ref, acc_ref, ers_ref, W_ref, b_ref, ew_ref, e1_ref, evs_ref):
        a = acc_ref[...]
        e1 = (e_ref[...] + a[0] + a[1]) / ers_ref[...]
        e1_ref[...] = e1
        ev = jnp.maximum(jnp.dot(e1, W_ref[...],
                                 preferred_element_type=jnp.float32)
                         + b_ref[...], 0.0)
        evs_ref[...] = ev * ew_ref[...]

    return _tc_call(
        body, (n // blk,),
        [_row_specs(n, blk, 0),
         pl.BlockSpec((2, blk, H), lambda i: (0, i, 0)),
         pl.BlockSpec((blk, 1), lambda i: (i, 0)),
         _full2d(H, H), _full2d(1, H),
         pl.BlockSpec((blk, 1), lambda i: (i, 0))],
        [_row_specs(n, blk, 0)] * 2,
        [jax.ShapeDtypeStruct((n, H), jnp.float32)] * 2,
        (e, eacc, ers, W, b, ew))


def _stage3(vA, vacc, nrs, W1, b1, nw, blk):
    n = vA.shape[0]

    def body(vA_ref, acc_ref, nrs_ref, W_ref, b_ref, nw_ref,
             v2_ref, vB_ref, ve2s_ref):
        a = acc_ref[...]
        v2 = (vA_ref[...] + a[0] + a[1]) / nrs_ref[...]
        v2_ref[...] = v2
        nwb = nw_ref[...]
        vB_ref[...] = v2 * nwb
        ve2 = jnp.maximum(
            (1.0 - BETA) * (jnp.dot(v2, W_ref[...],
                                    preferred_element_type=jnp.float32)
                            + b_ref[...]) + BETA * v2, 0.0)
        ve2s_ref[...] = ve2 * nwb

    return _tc_call(
        body, (n // blk,),
        [_row_specs(n, blk, 0),
         pl.BlockSpec((2, blk, H), lambda i: (0, i, 0)),
         pl.BlockSpec((blk, 1), lambda i: (i, 0)),
         _full2d(H, H), _full2d(1, H),
         pl.BlockSpec((blk, 1), lambda i: (i, 0))],
        [_row_specs(n, blk, 0)] * 3,
        [jax.ShapeDtypeStruct((n, H), jnp.float32)] * 3,
        (vA, vacc, nrs, W1, b1, nw))


def _stage4(e1, eacc2, ers, W, b, ew, blk):
    n = e1.shape[0]

    def body(e1_ref, acc_ref, ers_ref, W_ref, b_ref, ew_ref,
             e2_ref, ev2s_ref):
        a = acc_ref[...]
        e1 = e1_ref[...]
        e2a = (e1 + a[0] + a[1]) / ers_ref[...]
        e2 = (1.0 - ALPHA) * e2a + ALPHA * e1
        e2_ref[...] = e2
        ev2 = jnp.maximum(
            (1.0 - BETA) * (jnp.dot(e2, W_ref[...],
                                    preferred_element_type=jnp.float32)
                            + b_ref[...]) + BETA * e2, 0.0)
        ev2s_ref[...] = ev2 * ew_ref[...]

    return _tc_call(
        body, (n // blk,),
        [_row_specs(n, blk, 0),
         pl.BlockSpec((2, blk, H), lambda i: (0, i, 0)),
         pl.BlockSpec((blk, 1), lambda i: (i, 0)),
         _full2d(H, H), _full2d(1, H),
         pl.BlockSpec((blk, 1), lambda i: (i, 0))],
        [_row_specs(n, blk, 0)] * 2,
        [jax.ShapeDtypeStruct((n, H), jnp.float32)] * 2,
        (e1, eacc2, ers, W, b, ew))


def _stage5(vB, vacc2, nrs, v2, W_cls, b_cls, blk):
    n = vB.shape[0]
    ncls = W_cls.shape[1]

    def body(vB_ref, acc_ref, nrs_ref, v2_ref, W_ref, b_ref,
             vout_ref, pred_ref):
        a = acc_ref[...]
        v3 = (vB_ref[...] + a[0] + a[1]) / nrs_ref[...]
        vout = (1.0 - ALPHA) * v3 + ALPHA * v2_ref[...]
        vout_ref[...] = vout
        pred_ref[...] = jnp.dot(vout, W_ref[...],
                                preferred_element_type=jnp.float32) + b_ref[...]

    return _tc_call(
        body, (n // blk,),
        [_row_specs(n, blk, 0),
         pl.BlockSpec((2, blk, H), lambda i: (0, i, 0)),
         pl.BlockSpec((blk, 1), lambda i: (i, 0)),
         _row_specs(n, blk, 0),
         _full2d(H, ncls), _full2d(1, ncls)],
        [_row_specs(n, blk, 0), pl.BlockSpec((blk, ncls), lambda i: (i, 0))],
        [jax.ShapeDtypeStruct((n, H), jnp.float32),
         jax.ShapeDtypeStruct((n, ncls), jnp.float32)],
        (vB, vacc2, nrs, v2, W_cls, b_cls))


def kernel(v, e, W_vtx, b_vtx, W_v2e0, b_v2e0, W_e2v0, b_e2v0,
           W_v2e1, b_v2e1, W_e2v1, b_e2v1, W_cls, b_cls,
           vidx, eidx, n_weight, e_weight, n_reg_weight, e_reg_weight,
           n_reg_sum, e_reg_sum):
    NV = v.shape[0]
    NE = e.shape[0]
    E = vidx.shape[0]

    info = plsc.get_sparse_core_info()
    NW = info.num_cores * info.num_subcores
    KC, NGC = 40, 10
    vidx2 = vidx.reshape(NW, NGC, -1, KC)
    eidx2 = eidx.reshape(NW, NGC, -1, KC)
    nrw2 = n_reg_weight.reshape(NW, NGC, -1, KC)
    erw2 = e_reg_weight.reshape(NW, NGC, -1, KC)
    b_vtx2 = b_vtx.reshape(1, H)
    b_v2e0_2 = b_v2e0.reshape(1, H)
    b_e2v0_2 = b_e2v0.reshape(1, H)
    b_v2e1_2 = b_v2e1.reshape(1, H)
    b_e2v1_2 = b_e2v1.reshape(1, H)
    b_cls2 = b_cls.reshape(1, -1)

    blk_v = 1000
    blk_e = 1000

    sc_v2e = _make_sc_pass(NV, NE, E, packed=False)
    sc_e2v = _make_sc_pass(NE, NV, E, packed=False)

    def _pack(t):
        n = t.shape[0]
        return lax.bitcast_convert_type(t.reshape(n, HW, 2), jnp.int32)

    def _unperm(p):
        n = p.shape[1]
        return p.reshape(2, n, H // 32, 2, 16).swapaxes(3, 4).reshape(
            2, n, H)

    vA, ves = _stage1(v, W_vtx, b_vtx2, W_v2e0, b_v2e0_2, n_weight, blk_v)
    eacc = sc_v2e(ves, vidx2, eidx2, nrw2)
    e1, evs = _stage2(e, eacc, e_reg_sum, W_e2v0, b_e2v0_2, e_weight, blk_e)
    vacc = sc_e2v(evs, eidx2, vidx2, erw2)
    v2, vB, ve2s = _stage3(vA, vacc, n_reg_sum, W_v2e1, b_v2e1_2,
                           n_weight, blk_v)
    eacc2 = sc_v2e(ve2s, vidx2, eidx2, nrw2)
    e2, ev2s = _stage4(e1, eacc2, e_reg_sum, W_e2v1, b_e2v1_2,
                       e_weight, blk_e)
    vacc2 = sc_e2v(ev2s, eidx2, vidx2, erw2)
    v_out, pred = _stage5(vB, vacc2, n_reg_sum, v2, W_cls, b_cls2, blk_v)

    return (v_out, e2, pred)
